# Initial kernel scaffold; baseline (speedup 1.0000x reference)
import jax
import jax.numpy as jnp
from jax.experimental import pallas as pl


def kernel(xyz, x):
    raise NotImplementedError("write your pallas kernel here")


# speedup vs baseline: 16.7458x; 16.7458x over previous
import functools

import jax
import jax.numpy as jnp
from jax import lax
from jax.experimental import pallas as pl
from jax.experimental.pallas import tpu as pltpu
from jax.experimental.pallas import tpu_sc as plsc

B = 8
N = 4096
C = 128
G = 1024
K = 24
NW = 32


def _fps_body(xyzT_ref, idx_ref, cx_ref, cy_ref, cz_ref, dist_ref):
    xb = xyzT_ref[0]
    yb = xyzT_ref[1]
    zb = xyzT_ref[2]
    dist_ref[:] = jnp.full((B, N), 1e10, jnp.float32)
    boff = lax.broadcasted_iota(jnp.int32, (B, 1), 0) * N
    iota = lax.broadcasted_iota(jnp.int32, (B, N), 1)
    lane = lax.broadcasted_iota(jnp.int32, (B, 128), 1)
    zf = jnp.zeros((B, 128), jnp.float32)
    zi = jnp.zeros((B, 128), jnp.int32)

    def inner(j, carry):
        cx, cy, cz, fi, vidx, vcx, vcy, vcz = carry
        sel = lane == j
        vidx = jnp.where(sel, fi + boff, vidx)
        vcx = jnp.where(sel, cx, vcx)
        vcy = jnp.where(sel, cy, vcy)
        vcz = jnp.where(sel, cz, vcz)
        dx = xb - cx
        dy = yb - cy
        dz = zb - cz
        d = (dx * dx + dy * dy) + dz * dz
        dist = jnp.minimum(dist_ref[:], d)
        dist_ref[:] = dist
        m = jnp.max(dist, axis=1, keepdims=True)
        fi2 = jnp.min(jnp.where(dist == m, iota, jnp.int32(N)),
                      axis=1, keepdims=True)
        msk = iota == fi2
        cx2 = jnp.sum(jnp.where(msk, xb, 0.0), axis=1, keepdims=True)
        cy2 = jnp.sum(jnp.where(msk, yb, 0.0), axis=1, keepdims=True)
        cz2 = jnp.sum(jnp.where(msk, zb, 0.0), axis=1, keepdims=True)
        return (cx2, cy2, cz2, fi2, vidx, vcx, vcy, vcz)

    def outer(o, carry):
        carry = lax.fori_loop(0, 128, inner, carry)
        off = pl.multiple_of(o * 128, 128)
        idx_ref[:, pl.ds(off, 128)] = carry[4]
        cx_ref[:, pl.ds(off, 128)] = carry[5]
        cy_ref[:, pl.ds(off, 128)] = carry[6]
        cz_ref[:, pl.ds(off, 128)] = carry[7]
        return carry

    init = (xb[:, 0:1], yb[:, 0:1], zb[:, 0:1], jnp.zeros((B, 1), jnp.int32),
            zi, zf, zf, zf)
    lax.fori_loop(0, G // 128, outer, init)


_RQ = 256


def _knn_body(xb_ref, yb_ref, zb_ref, qx_ref, qy_ref, qz_ref, idx_ref, d_ref):
    b = pl.program_id(0)
    xb = xb_ref[0]
    yb = yb_ref[0]
    zb = zb_ref[0]
    pp = (xb * xb + yb * yb) + zb * zb
    qx = qx_ref[:]
    qy = qy_ref[:]
    qz = qz_ref[:]
    qq = (qx * qx + qy * qy) + qz * qz

    def r16(v):
        return v.astype(jnp.bfloat16).astype(jnp.float32)

    qp = (r16(qx) * r16(xb) + r16(qy) * r16(yb)) + r16(qz) * r16(zb)
    d_ref[:] = (qq + pp) - 2.0 * qp
    iota = lax.broadcasted_iota(jnp.int32, (_RQ, N), 1)
    boff = b * N

    for t in range(K):
        d = d_ref[:]
        m = jnp.min(d, axis=1, keepdims=True)
        j = jnp.min(jnp.where(d == m, iota, jnp.int32(N)),
                    axis=1, keepdims=True)
        idx_ref[:, t:t + 1] = j + boff
        d_ref[:] = jnp.where(iota == j, jnp.float32(3.0e38), d)


_CH = 256


def _sc_gather_body(x_tab, xc_t, yc_t, zc_t, fps_idx, knn_idx,
                    lc_x_out, ox_out, oy_out, oz_out, knn_x_out,
                    idx_a, rows_a, idx_b,
                    ox_v, oy_v, oz_v, rows_c, sem):
    wid = lax.axis_index("s") * 2 + lax.axis_index("c")
    na = (B * G) // NW
    base_a = wid * na
    pltpu.sync_copy(fps_idx.at[pl.ds(base_a, na)], idx_a)
    pltpu.async_copy(x_tab.at[idx_a], rows_a, sem).wait()
    pltpu.sync_copy(rows_a, lc_x_out.at[pl.ds(base_a, na)])

    nb = (B * G * K) // NW
    base_b = wid * nb
    pltpu.sync_copy(knn_idx.at[pl.ds(base_b, nb)], idx_b)

    pltpu.async_copy(xc_t.at[idx_b], ox_v, sem).wait()
    pltpu.async_copy(yc_t.at[idx_b], oy_v, sem).wait()
    pltpu.async_copy(zc_t.at[idx_b], oz_v, sem).wait()
    pltpu.sync_copy(ox_v, ox_out.at[pl.ds(base_b, nb)])
    pltpu.sync_copy(oy_v, oy_out.at[pl.ds(base_b, nb)])
    pltpu.sync_copy(oz_v, oz_out.at[pl.ds(base_b, nb)])

    def chunk_x(c, _):
        off = c * _CH
        pltpu.async_copy(x_tab.at[idx_b.at[pl.ds(off, _CH)]],
                         rows_c, sem).wait()
        pltpu.sync_copy(rows_c, knn_x_out.at[pl.ds(base_b + off, _CH)])
        return 0

    lax.fori_loop(0, nb // _CH, chunk_x, 0)


def _make_sc_gather():
    nb = (B * G * K) // NW
    na = (B * G) // NW
    mesh = plsc.VectorSubcoreMesh(core_axis_name="c", subcore_axis_name="s")
    return pl.kernel(
        _sc_gather_body,
        mesh=mesh,
        out_type=[
            jax.ShapeDtypeStruct((B * G, C), jnp.float32),
            jax.ShapeDtypeStruct((B * G * K,), jnp.float32),
            jax.ShapeDtypeStruct((B * G * K,), jnp.float32),
            jax.ShapeDtypeStruct((B * G * K,), jnp.float32),
            jax.ShapeDtypeStruct((B * G * K, C), jnp.float32),
        ],
        scratch_types=[
            pltpu.VMEM((na,), jnp.int32),
            pltpu.VMEM((na, C), jnp.float32),
            pltpu.VMEM((nb,), jnp.int32),
            pltpu.VMEM((nb,), jnp.float32),
            pltpu.VMEM((nb,), jnp.float32),
            pltpu.VMEM((nb,), jnp.float32),
            pltpu.VMEM((_CH, C), jnp.float32),
            pltpu.SemaphoreType.DMA,
        ],
    )


def kernel(xyz, x):
    xyzT = jnp.transpose(xyz, (2, 0, 1))

    fps_idx, cx, cy, cz = pl.pallas_call(
        _fps_body,
        out_shape=[
            jax.ShapeDtypeStruct((B, G), jnp.int32),
            jax.ShapeDtypeStruct((B, G), jnp.float32),
            jax.ShapeDtypeStruct((B, G), jnp.float32),
            jax.ShapeDtypeStruct((B, G), jnp.float32),
        ],
        scratch_shapes=[pltpu.VMEM((B, N), jnp.float32)],
    )(xyzT)

    lc_xyz = jnp.stack([cx, cy, cz], axis=-1)

    nsteps = (B * G) // _RQ
    qx = cx.reshape(B * G, 1)
    qy = cy.reshape(B * G, 1)
    qz = cz.reshape(B * G, 1)
    steps_per_b = G // _RQ
    xb3 = xyzT[0].reshape(B, 1, N)
    yb3 = xyzT[1].reshape(B, 1, N)
    zb3 = xyzT[2].reshape(B, 1, N)
    knn_idx = pl.pallas_call(
        _knn_body,
        grid=(B, steps_per_b),
        in_specs=[
            pl.BlockSpec((1, 1, N), lambda b, s: (b, 0, 0)),
            pl.BlockSpec((1, 1, N), lambda b, s: (b, 0, 0)),
            pl.BlockSpec((1, 1, N), lambda b, s: (b, 0, 0)),
            pl.BlockSpec((_RQ, 1), lambda b, s: (b * steps_per_b + s, 0)),
            pl.BlockSpec((_RQ, 1), lambda b, s: (b * steps_per_b + s, 0)),
            pl.BlockSpec((_RQ, 1), lambda b, s: (b * steps_per_b + s, 0)),
        ],
        out_specs=pl.BlockSpec((_RQ, K), lambda b, s: (b * steps_per_b + s, 0)),
        out_shape=jax.ShapeDtypeStruct((B * G, K), jnp.int32),
        scratch_shapes=[pltpu.VMEM((_RQ, N), jnp.float32)],
    )(xb3, yb3, zb3, qx, qy, qz)
    del nsteps

    x_tab = x.reshape(B * N, C)
    xc_t = xyzT[0].reshape(B * N)
    yc_t = xyzT[1].reshape(B * N)
    zc_t = xyzT[2].reshape(B * N)
    fps_idx_flat = fps_idx.reshape(B * G)
    knn_idx_flat = knn_idx.reshape(B * G * K)

    lc_x_flat, ox, oy, oz, knn_x_flat = _make_sc_gather()(
        x_tab, xc_t, yc_t, zc_t, fps_idx_flat, knn_idx_flat)

    lc_x = lc_x_flat.reshape(B, G, C)
    knn_xyz = jnp.stack([ox, oy, oz], axis=-1).reshape(B, G, K, 3)
    knn_x = knn_x_flat.reshape(B, G, K, C)
    return (lc_xyz, lc_x, knn_xyz, knn_x)


# speedup vs baseline: 20.0718x; 1.1986x over previous
import functools

import jax
import jax.numpy as jnp
from jax import lax
from jax.experimental import pallas as pl
from jax.experimental.pallas import tpu as pltpu
from jax.experimental.pallas import tpu_sc as plsc

B = 8
N = 4096
C = 128
G = 1024
K = 24
NW = 32


def _fps_body(xyzT_ref, idx_ref, cx_ref, cy_ref, cz_ref, dist_ref):
    xb = xyzT_ref[0]
    yb = xyzT_ref[1]
    zb = xyzT_ref[2]
    dist_ref[:] = jnp.full((B, N), 1e10, jnp.float32)
    boff = lax.broadcasted_iota(jnp.int32, (B, 1), 0) * N
    iota = lax.broadcasted_iota(jnp.int32, (B, N), 1).astype(jnp.float32)
    lane = lax.broadcasted_iota(jnp.int32, (B, 128), 1)
    zf = jnp.zeros((B, 128), jnp.float32)
    zi = jnp.zeros((B, 128), jnp.int32)

    def inner(j, carry):
        cx, cy, cz, fi, vidx, vcx, vcy, vcz = carry
        sel = lane == j
        vidx = jnp.where(sel, fi + boff, vidx)
        vcx = jnp.where(sel, cx, vcx)
        vcy = jnp.where(sel, cy, vcy)
        vcz = jnp.where(sel, cz, vcz)
        dx = xb - cx
        dy = yb - cy
        dz = zb - cz
        d = (dx * dx + dy * dy) + dz * dz
        dist = jnp.minimum(dist_ref[:], d)
        dist_ref[:] = dist
        rd, ri, rx, ry, rz = dist, iota, xb, yb, zb
        w = N // 2
        while w >= 128:
            da, db = rd[:, :w], rd[:, w:]
            ia, ib = ri[:, :w], ri[:, w:]
            ta = (da > db) | ((da == db) & (ia < ib))
            rd = jnp.where(ta, da, db)
            ri = jnp.where(ta, ia, ib)
            rx = jnp.where(ta, rx[:, :w], rx[:, w:])
            ry = jnp.where(ta, ry[:, :w], ry[:, w:])
            rz = jnp.where(ta, rz[:, :w], rz[:, w:])
            w //= 2
        m = jnp.max(rd, axis=1, keepdims=True)
        fi_f = jnp.min(jnp.where(rd == m, ri, jnp.float32(N)),
                       axis=1, keepdims=True)
        msk = ri == fi_f
        cx2 = jnp.sum(jnp.where(msk, rx, 0.0), axis=1, keepdims=True)
        cy2 = jnp.sum(jnp.where(msk, ry, 0.0), axis=1, keepdims=True)
        cz2 = jnp.sum(jnp.where(msk, rz, 0.0), axis=1, keepdims=True)
        fi2 = fi_f.astype(jnp.int32)
        return (cx2, cy2, cz2, fi2, vidx, vcx, vcy, vcz)

    def outer(o, carry):
        carry = lax.fori_loop(0, 128, inner, carry)
        off = pl.multiple_of(o * 128, 128)
        idx_ref[:, pl.ds(off, 128)] = carry[4]
        cx_ref[:, pl.ds(off, 128)] = carry[5]
        cy_ref[:, pl.ds(off, 128)] = carry[6]
        cz_ref[:, pl.ds(off, 128)] = carry[7]
        return carry

    init = (xb[:, 0:1], yb[:, 0:1], zb[:, 0:1], jnp.zeros((B, 1), jnp.int32),
            zi, zf, zf, zf)
    lax.fori_loop(0, G // 128, outer, init)


_RQ = 256


def _knn_body(xb_ref, yb_ref, zb_ref, qx_ref, qy_ref, qz_ref, idx_ref, d_ref):
    b = pl.program_id(0)
    xb = xb_ref[0]
    yb = yb_ref[0]
    zb = zb_ref[0]
    pp = (xb * xb + yb * yb) + zb * zb
    qx = qx_ref[:]
    qy = qy_ref[:]
    qz = qz_ref[:]
    qq = (qx * qx + qy * qy) + qz * qz

    def r16(v):
        return v.astype(jnp.bfloat16).astype(jnp.float32)

    qp = (r16(qx) * r16(xb) + r16(qy) * r16(yb)) + r16(qz) * r16(zb)
    d_ref[:] = (qq + pp) - 2.0 * qp
    iota = lax.broadcasted_iota(jnp.int32, (_RQ, N), 1).astype(jnp.float32)
    boff = b * N

    for t in range(K):
        d = d_ref[:]
        m = jnp.min(d, axis=1, keepdims=True)
        j = jnp.min(jnp.where(d == m, iota, jnp.float32(N)),
                    axis=1, keepdims=True)
        idx_ref[:, t:t + 1] = j.astype(jnp.int32) + boff
        d_ref[:] = jnp.where(iota == j, jnp.float32(3.0e38), d)


_CH = 256


def _sc_gather_body(x_tab, xc_t, yc_t, zc_t, fps_idx, knn_idx,
                    lc_x_out, ox_out, oy_out, oz_out, knn_x_out,
                    idx_a, rows_a, idx_b,
                    ox_v, oy_v, oz_v, rows_c, sem):
    wid = lax.axis_index("s") * 2 + lax.axis_index("c")
    na = (B * G) // NW
    base_a = wid * na
    pltpu.sync_copy(fps_idx.at[pl.ds(base_a, na)], idx_a)
    pltpu.async_copy(x_tab.at[idx_a], rows_a, sem).wait()
    pltpu.sync_copy(rows_a, lc_x_out.at[pl.ds(base_a, na)])

    nb = (B * G * K) // NW
    base_b = wid * nb
    pltpu.sync_copy(knn_idx.at[pl.ds(base_b, nb)], idx_b)

    pltpu.async_copy(xc_t.at[idx_b], ox_v, sem).wait()
    pltpu.async_copy(yc_t.at[idx_b], oy_v, sem).wait()
    pltpu.async_copy(zc_t.at[idx_b], oz_v, sem).wait()
    pltpu.sync_copy(ox_v, ox_out.at[pl.ds(base_b, nb)])
    pltpu.sync_copy(oy_v, oy_out.at[pl.ds(base_b, nb)])
    pltpu.sync_copy(oz_v, oz_out.at[pl.ds(base_b, nb)])

    def chunk_x(c, _):
        off = c * _CH
        pltpu.async_copy(x_tab.at[idx_b.at[pl.ds(off, _CH)]],
                         rows_c, sem).wait()
        pltpu.sync_copy(rows_c, knn_x_out.at[pl.ds(base_b + off, _CH)])
        return 0

    lax.fori_loop(0, nb // _CH, chunk_x, 0)


def _make_sc_gather():
    nb = (B * G * K) // NW
    na = (B * G) // NW
    mesh = plsc.VectorSubcoreMesh(core_axis_name="c", subcore_axis_name="s")
    return pl.kernel(
        _sc_gather_body,
        mesh=mesh,
        out_type=[
            jax.ShapeDtypeStruct((B * G, C), jnp.float32),
            jax.ShapeDtypeStruct((B * G * K,), jnp.float32),
            jax.ShapeDtypeStruct((B * G * K,), jnp.float32),
            jax.ShapeDtypeStruct((B * G * K,), jnp.float32),
            jax.ShapeDtypeStruct((B * G * K, C), jnp.float32),
        ],
        scratch_types=[
            pltpu.VMEM((na,), jnp.int32),
            pltpu.VMEM((na, C), jnp.float32),
            pltpu.VMEM((nb,), jnp.int32),
            pltpu.VMEM((nb,), jnp.float32),
            pltpu.VMEM((nb,), jnp.float32),
            pltpu.VMEM((nb,), jnp.float32),
            pltpu.VMEM((_CH, C), jnp.float32),
            pltpu.SemaphoreType.DMA,
        ],
    )


def kernel(xyz, x):
    xyzT = jnp.transpose(xyz, (2, 0, 1))

    fps_idx, cx, cy, cz = pl.pallas_call(
        _fps_body,
        out_shape=[
            jax.ShapeDtypeStruct((B, G), jnp.int32),
            jax.ShapeDtypeStruct((B, G), jnp.float32),
            jax.ShapeDtypeStruct((B, G), jnp.float32),
            jax.ShapeDtypeStruct((B, G), jnp.float32),
        ],
        scratch_shapes=[pltpu.VMEM((B, N), jnp.float32)],
    )(xyzT)

    lc_xyz = jnp.stack([cx, cy, cz], axis=-1)

    nsteps = (B * G) // _RQ
    qx = cx.reshape(B * G, 1)
    qy = cy.reshape(B * G, 1)
    qz = cz.reshape(B * G, 1)
    steps_per_b = G // _RQ
    xb3 = xyzT[0].reshape(B, 1, N)
    yb3 = xyzT[1].reshape(B, 1, N)
    zb3 = xyzT[2].reshape(B, 1, N)
    knn_idx = pl.pallas_call(
        _knn_body,
        grid=(B, steps_per_b),
        in_specs=[
            pl.BlockSpec((1, 1, N), lambda b, s: (b, 0, 0)),
            pl.BlockSpec((1, 1, N), lambda b, s: (b, 0, 0)),
            pl.BlockSpec((1, 1, N), lambda b, s: (b, 0, 0)),
            pl.BlockSpec((_RQ, 1), lambda b, s: (b * steps_per_b + s, 0)),
            pl.BlockSpec((_RQ, 1), lambda b, s: (b * steps_per_b + s, 0)),
            pl.BlockSpec((_RQ, 1), lambda b, s: (b * steps_per_b + s, 0)),
        ],
        out_specs=pl.BlockSpec((_RQ, K), lambda b, s: (b * steps_per_b + s, 0)),
        out_shape=jax.ShapeDtypeStruct((B * G, K), jnp.int32),
        scratch_shapes=[pltpu.VMEM((_RQ, N), jnp.float32)],
    )(xb3, yb3, zb3, qx, qy, qz)
    del nsteps

    x_tab = x.reshape(B * N, C)
    xc_t = xyzT[0].reshape(B * N)
    yc_t = xyzT[1].reshape(B * N)
    zc_t = xyzT[2].reshape(B * N)
    fps_idx_flat = fps_idx.reshape(B * G)
    knn_idx_flat = knn_idx.reshape(B * G * K)

    lc_x_flat, ox, oy, oz, knn_x_flat = _make_sc_gather()(
        x_tab, xc_t, yc_t, zc_t, fps_idx_flat, knn_idx_flat)

    lc_x = lc_x_flat.reshape(B, G, C)
    knn_xyz = jnp.stack([ox, oy, oz], axis=-1).reshape(B, G, K, 3)
    knn_x = knn_x_flat.reshape(B, G, K, C)
    return (lc_xyz, lc_x, knn_xyz, knn_x)
